# baseline (speedup 1.0000x reference)
import functools

import jax
import jax.numpy as jnp
from jax.experimental import pallas as pl
from jax.experimental.pallas import tpu as pltpu

_BLK = 2048
_NGRAPH = 256


def _embed_gcn_body(x_ref, emb_W_ref, emb_b_ref, gcn_W_ref, h_ref):
    x_blk = x_ref[...]
    e = jnp.dot(x_blk[:, :2], emb_W_ref[...],
                preferred_element_type=jnp.float32) + emb_b_ref[...]
    x64 = jnp.concatenate([e, x_blk[:, 2:]], axis=1)
    h_ref[...] = jnp.dot(x64, gcn_W_ref[...],
                         preferred_element_type=jnp.float32)


def _head_body(agg_ref, batch_ref, gcn_b_ref, fc1_W_ref, fc1_b_ref,
               fc2_W_ref, fc2_b_ref, out_ref, pooled_acc):
    i = pl.program_id(0)
    n_blocks = pl.num_programs(0)

    @pl.when(i == 0)
    def _init():
        pooled_acc[...] = jnp.zeros_like(pooled_acc)

    act = agg_ref[...] + gcn_b_ref[...]
    act = 0.5 * act * (1.0 + jax.lax.erf(act / jnp.sqrt(2.0).astype(jnp.float32)))

    batch_col = batch_ref[...].reshape(-1, 1)
    graph_iota = jax.lax.broadcasted_iota(
        jnp.int32, (1, _NGRAPH), 1).astype(jnp.float32)
    onehot = (batch_col == graph_iota).astype(jnp.float32)
    pooled_acc[...] += jnp.dot(onehot.T, act,
                               preferred_element_type=jnp.float32)

    @pl.when(i == n_blocks - 1)
    def _finish():
        pooled = pooled_acc[...]
        h2 = jnp.dot(pooled, fc1_W_ref[...],
                     preferred_element_type=jnp.float32) + fc1_b_ref[...]
        h2 = 0.5 * h2 * (1.0 + jax.lax.erf(h2 / jnp.sqrt(2.0).astype(jnp.float32)))
        out_ref[...] = jnp.dot(h2, fc2_W_ref[...],
                               preferred_element_type=jnp.float32) + fc2_b_ref[...]


@functools.partial(jax.jit, static_argnames=("n_pad",))
def _embed_gcn(x_p, emb_W, emb_b, gcn_W, n_pad):
    n_blocks = n_pad // _BLK
    f_in = x_p.shape[1]
    emb_out = emb_W.shape[1]
    return pl.pallas_call(
        _embed_gcn_body,
        grid=(n_blocks,),
        in_specs=[
            pl.BlockSpec((_BLK, f_in), lambda i: (i, 0)),
            pl.BlockSpec((2, emb_out), lambda i: (0, 0)),
            pl.BlockSpec((emb_out,), lambda i: (0,)),
            pl.BlockSpec((f_in - 2 + emb_out, gcn_W.shape[1]), lambda i: (0, 0)),
        ],
        out_specs=pl.BlockSpec((_BLK, gcn_W.shape[1]), lambda i: (i, 0)),
        out_shape=jax.ShapeDtypeStruct((n_pad, gcn_W.shape[1]), jnp.float32),
    )(x_p, emb_W, emb_b, gcn_W)


@functools.partial(jax.jit, static_argnames=("n_pad",))
def _head(agg_p, batch_f, gcn_b, fc1_W, fc1_b, fc2_W, fc2_b, n_pad):
    n_blocks = n_pad // _BLK
    nhid = agg_p.shape[1]
    return pl.pallas_call(
        _head_body,
        grid=(n_blocks,),
        in_specs=[
            pl.BlockSpec((_BLK, nhid), lambda i: (i, 0)),
            pl.BlockSpec((1, 1, _BLK), lambda i: (i, 0, 0)),
            pl.BlockSpec((nhid,), lambda i: (0,)),
            pl.BlockSpec((nhid, nhid), lambda i: (0, 0)),
            pl.BlockSpec((nhid,), lambda i: (0,)),
            pl.BlockSpec((nhid, 1), lambda i: (0, 0)),
            pl.BlockSpec((1,), lambda i: (0,)),
        ],
        out_specs=pl.BlockSpec((_NGRAPH, 1), lambda i: (0, 0)),
        out_shape=jax.ShapeDtypeStruct((_NGRAPH, 1), jnp.float32),
        scratch_shapes=[pltpu.VMEM((_NGRAPH, nhid), jnp.float32)],
    )(agg_p, batch_f, gcn_b, fc1_W, fc1_b, fc2_W, fc2_b)


def kernel(x, edge_index, batch, known_mask, unk_mask, obs_mask, msg_weights,
           emb_W, emb_b, gcn_W, gcn_b, fc1_W, fc1_b, fc2_W, fc2_b):
    n = x.shape[0]
    e = edge_index.shape[1]
    n_pad = ((n + _BLK - 1) // _BLK) * _BLK

    sm = jax.nn.softmax(msg_weights)
    ew = jnp.ones((e,), jnp.float32)
    ew = ew.at[known_mask].set(sm[0])
    ew = ew.at[unk_mask].set(sm[1])
    ew = ew.at[obs_mask].set(sm[2])

    x_p = jnp.zeros((n_pad, x.shape[1]), jnp.float32).at[:n].set(x)
    h = _embed_gcn(x_p, emb_W, emb_b, gcn_W, n_pad)[:n]

    row = edge_index[0]
    col = edge_index[1]
    loop = jnp.arange(n, dtype=row.dtype)
    row_f = jnp.concatenate([row, loop])
    col_f = jnp.concatenate([col, loop])
    ew_f = jnp.concatenate([ew, jnp.ones((n,), jnp.float32)])
    deg = jax.ops.segment_sum(ew_f, col_f, num_segments=n)
    dinv = jnp.where(deg > 0, jax.lax.rsqrt(jnp.maximum(deg, 1e-12)), 0.0)
    norm = dinv[row_f] * ew_f * dinv[col_f]
    msg = norm[:, None] * jnp.take(h, row_f, axis=0)
    agg = jax.ops.segment_sum(msg, col_f, num_segments=n)

    agg_p = jnp.zeros((n_pad, agg.shape[1]), jnp.float32).at[:n].set(agg)
    batch_f = jnp.full((n_pad,), float(_NGRAPH + 1), jnp.float32)
    batch_f = batch_f.at[:n].set(batch.astype(jnp.float32))
    batch_f = batch_f.reshape(n_pad // _BLK, 1, _BLK)
    return _head(agg_p, batch_f, gcn_b, fc1_W, fc1_b, fc2_W, fc2_b, n_pad)
